# baseline (speedup 1.0000x reference)
import functools

import jax
import jax.numpy as jnp
from jax import lax
from jax.experimental import pallas as pl
from jax.experimental.pallas import tpu as pltpu
from jax.experimental.pallas import tpu_sc as plsc

DIM_ = 256
NCODES_ = 8192
NTOK_ = 8192
TOK_BLK_ = 512
NBLK_ = NTOK_ // TOK_BLK_


_WIN_BOUNDS_ = ((0, 2816), (2816, 5632), (5632, 8192))


def _rne_bf16(v):
    b = jax.lax.bitcast_convert_type(v, jnp.uint32)
    r = (b + jnp.uint32(0x7FFF) + ((b >> 16) & jnp.uint32(1))) \
        & jnp.uint32(0xFFFF0000)
    return jax.lax.bitcast_convert_type(r, jnp.float32)


def _neg_rne_bf16(v):
    return _rne_bf16(v)


def _argmin_body(x_ref, e_ref, ind_ref, et_ref, e2_ref):
    i = pl.program_id(0)
    x = x_ref[...]
    e = e_ref[...]

    @pl.when(i == 0)
    def _prep():
        e2_ref[...] = jnp.sum(e * e, axis=0, keepdims=True)

    et_ref[...] = e_ref[:, pl.ds(i * TOK_BLK_, TOK_BLK_)].T

    f2 = jnp.sum(x * x, axis=1, keepdims=True)
    e2 = e2_ref[...]
    m = jax.lax.dot_general(
        x, e, (((1,), (0,)), ((), ())),
        preferred_element_type=jnp.float32)
    score = (f2 - 2.0 * m) + e2
    run_v = None
    run_i = None
    for (a, b) in _WIN_BOUNDS_:
        sub = score[:, a:b]
        mn = jnp.min(sub, axis=1, keepdims=True)
        io = jax.lax.broadcasted_iota(jnp.int32, sub.shape, 1) + jnp.int32(a)
        idx = jnp.min(jnp.where(sub == mn, io, jnp.int32(NCODES_)),
                      axis=1, keepdims=True)
        if run_v is None:
            run_v, run_i = mn, idx
        else:
            take = mn < run_v
            run_i = jnp.where(take, idx, run_i)
            run_v = jnp.minimum(run_v, mn)
        run_v = _neg_rne_bf16(run_v)
    ind_ref[0, 0, :] = run_i[:, 0]


def _compute_ind(x, embed):
    return pl.pallas_call(
        _argmin_body,
        grid=(NBLK_,),
        in_specs=[
            pl.BlockSpec((TOK_BLK_, DIM_), lambda i: (i, 0)),
            pl.BlockSpec((DIM_, NCODES_), lambda i: (0, 0)),
        ],
        out_specs=[
            pl.BlockSpec((1, 1, TOK_BLK_), lambda i: (i, 0, 0)),
            pl.BlockSpec((TOK_BLK_, DIM_), lambda i: (i, 0)),
        ],
        out_shape=[
            jax.ShapeDtypeStruct((NBLK_, 1, TOK_BLK_), jnp.int32),
            jax.ShapeDtypeStruct((NCODES_, DIM_), jnp.float32),
        ],
        scratch_shapes=[pltpu.VMEM((1, NCODES_), jnp.float32)],
        compiler_params=pltpu.CompilerParams(
            dimension_semantics=("arbitrary",),
        ),
    )(x, embed)


_NW_ = 32
_ROWS_PER_W_ = NTOK_ // _NW_
_IDX_CHUNK_ = 128


def _sc_gather_body(table_hbm, idx_hbm, out_hbm, idx_v, rows_v, sem):
    wid = lax.axis_index("s") * 2 + lax.axis_index("c")
    nchunks = _ROWS_PER_W_ // _IDX_CHUNK_
    base_row = wid * nchunks
    pltpu.sync_copy(idx_hbm.at[pl.ds(base_row, nchunks)], idx_v)
    copies = []
    for j in range(nchunks):
        copies.append(pltpu.async_copy(
            table_hbm.at[idx_v.at[j]],
            rows_v.at[pl.ds(j * _IDX_CHUNK_, _IDX_CHUNK_)],
            sem))
    for c in copies:
        c.wait()
    pltpu.sync_copy(rows_v, out_hbm.at[pl.ds(wid * _ROWS_PER_W_, _ROWS_PER_W_)])


@functools.lru_cache(maxsize=1)
def _make_sc_gather():
    return pl.kernel(
        _sc_gather_body,
        out_type=jax.ShapeDtypeStruct((NTOK_, DIM_), jnp.float32),
        mesh=plsc.VectorSubcoreMesh(core_axis_name="c", subcore_axis_name="s"),
        scratch_types=[
            pltpu.VMEM((_ROWS_PER_W_ // _IDX_CHUNK_, _IDX_CHUNK_), jnp.int32),
            pltpu.VMEM((_ROWS_PER_W_, DIM_), jnp.float32),
            pltpu.SemaphoreType.DMA,
        ],
    )


def _finish_body(x_ref, q_ref, qst_ref, diff_ref):
    i = pl.program_id(0)
    x = x_ref[...]
    d = q_ref[...] - x
    qst_ref[...] = x + d

    @pl.when(i == 0)
    def _init():
        diff_ref[...] = jnp.zeros((1, 1), jnp.float32)

    diff_ref[...] += jnp.sum(d * d).reshape(1, 1)

    @pl.when(i == NBLK_ - 1)
    def _final():
        diff_ref[...] = diff_ref[...] * jnp.float32(1.0 / (NTOK_ * DIM_))


def _finish(x, q):
    return pl.pallas_call(
        _finish_body,
        grid=(NBLK_,),
        in_specs=[
            pl.BlockSpec((TOK_BLK_, DIM_), lambda i: (i, 0)),
            pl.BlockSpec((TOK_BLK_, DIM_), lambda i: (i, 0)),
        ],
        out_specs=[
            pl.BlockSpec((TOK_BLK_, DIM_), lambda i: (i, 0)),
            pl.BlockSpec((1, 1), lambda i: (0, 0)),
        ],
        out_shape=[
            jax.ShapeDtypeStruct((NTOK_, DIM_), jnp.float32),
            jax.ShapeDtypeStruct((1, 1), jnp.float32),
        ],
        compiler_params=pltpu.CompilerParams(
            dimension_semantics=("arbitrary",),
        ),
    )(x, q)


def kernel(input, embed):
    x = input.reshape(NTOK_, DIM_)
    ind3, table = _compute_ind(x, embed)
    ind = ind3.reshape(NTOK_)
    idx2 = ind.reshape(NTOK_ // _IDX_CHUNK_, _IDX_CHUNK_)
    q = _make_sc_gather()(table, idx2)
    qst, diff = _finish(x, q)
    return (qst.reshape(input.shape),
            diff.reshape(()),
            jnp.zeros(1, dtype=jnp.float32),
            ind.reshape(input.shape[:-1]))


# speedup vs baseline: 1.1028x; 1.1028x over previous
import functools

import jax
import jax.numpy as jnp
from jax import lax
from jax.experimental import pallas as pl
from jax.experimental.pallas import tpu as pltpu
from jax.experimental.pallas import tpu_sc as plsc

DIM_ = 256
NCODES_ = 8192
NTOK_ = 8192
TOK_BLK_ = 1024
NBLK_ = NTOK_ // TOK_BLK_


_WIN_BOUNDS_ = ((0, 2816), (2816, 5632), (5632, 8192))


def _rne_bf16(v):
    b = jax.lax.bitcast_convert_type(v, jnp.uint32)
    r = (b + jnp.uint32(0x7FFF) + ((b >> 16) & jnp.uint32(1))) \
        & jnp.uint32(0xFFFF0000)
    return jax.lax.bitcast_convert_type(r, jnp.float32)


def _neg_rne_bf16(v):
    return _rne_bf16(v)


def _argmin_body(x_ref, e_ref, ind_ref, et_ref, e2_ref):
    i = pl.program_id(0)
    x = x_ref[...]
    e = e_ref[...]

    @pl.when(i == 0)
    def _prep():
        e2_ref[...] = jnp.sum(e * e, axis=0, keepdims=True)

    et_ref[...] = e_ref[:, pl.ds(i * TOK_BLK_, TOK_BLK_)].T

    f2 = jnp.sum(x * x, axis=1, keepdims=True)
    e2 = e2_ref[...]
    m = jax.lax.dot_general(
        x, e, (((1,), (0,)), ((), ())),
        preferred_element_type=jnp.float32)
    score = (f2 - 2.0 * m) + e2
    run_v = None
    run_i = None
    for (a, b) in _WIN_BOUNDS_:
        sub = score[:, a:b]
        mn = jnp.min(sub, axis=1, keepdims=True)
        io = jax.lax.broadcasted_iota(jnp.int32, sub.shape, 1) + jnp.int32(a)
        idx = jnp.min(jnp.where(sub == mn, io, jnp.int32(NCODES_)),
                      axis=1, keepdims=True)
        if run_v is None:
            run_v, run_i = mn, idx
        else:
            take = mn < run_v
            run_i = jnp.where(take, idx, run_i)
            run_v = jnp.minimum(run_v, mn)
        run_v = _neg_rne_bf16(run_v)
    ind_ref[0, 0, :] = run_i[:, 0]


def _compute_ind(x, embed):
    return pl.pallas_call(
        _argmin_body,
        grid=(NBLK_,),
        in_specs=[
            pl.BlockSpec((TOK_BLK_, DIM_), lambda i: (i, 0)),
            pl.BlockSpec((DIM_, NCODES_), lambda i: (0, 0)),
        ],
        out_specs=[
            pl.BlockSpec((1, 1, TOK_BLK_), lambda i: (i, 0, 0)),
            pl.BlockSpec((TOK_BLK_, DIM_), lambda i: (i, 0)),
        ],
        out_shape=[
            jax.ShapeDtypeStruct((NBLK_, 1, TOK_BLK_), jnp.int32),
            jax.ShapeDtypeStruct((NCODES_, DIM_), jnp.float32),
        ],
        scratch_shapes=[pltpu.VMEM((1, NCODES_), jnp.float32)],
        compiler_params=pltpu.CompilerParams(
            dimension_semantics=("arbitrary",),
        ),
    )(x, embed)


_NW_ = 32
_ROWS_PER_W_ = NTOK_ // _NW_
_IDX_CHUNK_ = 128


def _sc_gather_body(table_hbm, idx_hbm, out_hbm, idx_v, rows_v, sem):
    wid = lax.axis_index("s") * 2 + lax.axis_index("c")
    nchunks = _ROWS_PER_W_ // _IDX_CHUNK_
    base_row = wid * nchunks
    pltpu.sync_copy(idx_hbm.at[pl.ds(base_row, nchunks)], idx_v)
    copies = []
    for j in range(nchunks):
        copies.append(pltpu.async_copy(
            table_hbm.at[idx_v.at[j]],
            rows_v.at[pl.ds(j * _IDX_CHUNK_, _IDX_CHUNK_)],
            sem))
    for c in copies:
        c.wait()
    pltpu.sync_copy(rows_v, out_hbm.at[pl.ds(wid * _ROWS_PER_W_, _ROWS_PER_W_)])


@functools.lru_cache(maxsize=1)
def _make_sc_gather():
    return pl.kernel(
        _sc_gather_body,
        out_type=jax.ShapeDtypeStruct((NTOK_, DIM_), jnp.float32),
        mesh=plsc.VectorSubcoreMesh(core_axis_name="c", subcore_axis_name="s"),
        scratch_types=[
            pltpu.VMEM((_ROWS_PER_W_ // _IDX_CHUNK_, _IDX_CHUNK_), jnp.int32),
            pltpu.VMEM((_ROWS_PER_W_, DIM_), jnp.float32),
            pltpu.SemaphoreType.DMA,
        ],
    )


def _finish_body(x_ref, q_ref, qst_ref, diff_ref):
    i = pl.program_id(0)
    x = x_ref[...]
    d = q_ref[...] - x
    qst_ref[...] = x + d

    @pl.when(i == 0)
    def _init():
        diff_ref[...] = jnp.zeros((1, 1), jnp.float32)

    diff_ref[...] += jnp.sum(d * d).reshape(1, 1)

    @pl.when(i == NBLK_ - 1)
    def _final():
        diff_ref[...] = diff_ref[...] * jnp.float32(1.0 / (NTOK_ * DIM_))


def _finish(x, q):
    return pl.pallas_call(
        _finish_body,
        grid=(NBLK_,),
        in_specs=[
            pl.BlockSpec((TOK_BLK_, DIM_), lambda i: (i, 0)),
            pl.BlockSpec((TOK_BLK_, DIM_), lambda i: (i, 0)),
        ],
        out_specs=[
            pl.BlockSpec((TOK_BLK_, DIM_), lambda i: (i, 0)),
            pl.BlockSpec((1, 1), lambda i: (0, 0)),
        ],
        out_shape=[
            jax.ShapeDtypeStruct((NTOK_, DIM_), jnp.float32),
            jax.ShapeDtypeStruct((1, 1), jnp.float32),
        ],
        compiler_params=pltpu.CompilerParams(
            dimension_semantics=("arbitrary",),
        ),
    )(x, q)


def kernel(input, embed):
    x = input.reshape(NTOK_, DIM_)
    ind3, table = _compute_ind(x, embed)
    ind = ind3.reshape(NTOK_)
    idx2 = ind.reshape(NTOK_ // _IDX_CHUNK_, _IDX_CHUNK_)
    q = _make_sc_gather()(table, idx2)
    qst, diff = _finish(x, q)
    return (qst.reshape(input.shape),
            diff.reshape(()),
            jnp.zeros(1, dtype=jnp.float32),
            ind.reshape(input.shape[:-1]))


# speedup vs baseline: 1.1968x; 1.0853x over previous
import functools

import jax
import jax.numpy as jnp
from jax import lax
from jax.experimental import pallas as pl
from jax.experimental.pallas import tpu as pltpu
from jax.experimental.pallas import tpu_sc as plsc

DIM_ = 256
NCODES_ = 8192
NTOK_ = 8192
TOK_BLK_ = 2048
NBLK_ = NTOK_ // TOK_BLK_


_WIN_BOUNDS_ = ((0, 2816), (2816, 5632), (5632, 8192))


def _rne_bf16(v):
    b = jax.lax.bitcast_convert_type(v, jnp.uint32)
    r = (b + jnp.uint32(0x7FFF) + ((b >> 16) & jnp.uint32(1))) \
        & jnp.uint32(0xFFFF0000)
    return jax.lax.bitcast_convert_type(r, jnp.float32)


def _neg_rne_bf16(v):
    return _rne_bf16(v)


def _argmin_body(x_ref, e_ref, ind_ref, et_ref, e2_ref):
    i = pl.program_id(0)
    x = x_ref[...]
    e = e_ref[...]

    @pl.when(i == 0)
    def _prep():
        e2_ref[...] = jnp.sum(e * e, axis=0, keepdims=True)

    et_ref[...] = e_ref[:, pl.ds(i * TOK_BLK_, TOK_BLK_)].T

    f2 = jnp.sum(x * x, axis=1, keepdims=True)
    e2 = e2_ref[...]
    m = jax.lax.dot_general(
        x, e, (((1,), (0,)), ((), ())),
        preferred_element_type=jnp.float32)
    score = (f2 - 2.0 * m) + e2
    run_v = None
    run_i = None
    for (a, b) in _WIN_BOUNDS_:
        sub = score[:, a:b]
        mn = jnp.min(sub, axis=1, keepdims=True)
        io = jax.lax.broadcasted_iota(jnp.int32, sub.shape, 1) + jnp.int32(a)
        idx = jnp.min(jnp.where(sub == mn, io, jnp.int32(NCODES_)),
                      axis=1, keepdims=True)
        if run_v is None:
            run_v, run_i = mn, idx
        else:
            take = mn < run_v
            run_i = jnp.where(take, idx, run_i)
            run_v = jnp.minimum(run_v, mn)
        run_v = _neg_rne_bf16(run_v)
    ind_ref[0, 0, :] = run_i[:, 0]


def _compute_ind(x, embed):
    return pl.pallas_call(
        _argmin_body,
        grid=(NBLK_,),
        in_specs=[
            pl.BlockSpec((TOK_BLK_, DIM_), lambda i: (i, 0)),
            pl.BlockSpec((DIM_, NCODES_), lambda i: (0, 0)),
        ],
        out_specs=[
            pl.BlockSpec((1, 1, TOK_BLK_), lambda i: (i, 0, 0)),
            pl.BlockSpec((TOK_BLK_, DIM_), lambda i: (i, 0)),
        ],
        out_shape=[
            jax.ShapeDtypeStruct((NBLK_, 1, TOK_BLK_), jnp.int32),
            jax.ShapeDtypeStruct((NCODES_, DIM_), jnp.float32),
        ],
        scratch_shapes=[pltpu.VMEM((1, NCODES_), jnp.float32)],
        compiler_params=pltpu.CompilerParams(
            dimension_semantics=("arbitrary",),
        ),
    )(x, embed)


_NW_ = 32
_ROWS_PER_W_ = NTOK_ // _NW_
_IDX_CHUNK_ = 128


def _sc_gather_body(table_hbm, idx_hbm, out_hbm, idx_v, rows_v, sem):
    wid = lax.axis_index("s") * 2 + lax.axis_index("c")
    nchunks = _ROWS_PER_W_ // _IDX_CHUNK_
    base_row = wid * nchunks
    pltpu.sync_copy(idx_hbm.at[pl.ds(base_row, nchunks)], idx_v)
    copies = []
    for j in range(nchunks):
        copies.append(pltpu.async_copy(
            table_hbm.at[idx_v.at[j]],
            rows_v.at[pl.ds(j * _IDX_CHUNK_, _IDX_CHUNK_)],
            sem))
    for c in copies:
        c.wait()
    pltpu.sync_copy(rows_v, out_hbm.at[pl.ds(wid * _ROWS_PER_W_, _ROWS_PER_W_)])


@functools.lru_cache(maxsize=1)
def _make_sc_gather():
    return pl.kernel(
        _sc_gather_body,
        out_type=jax.ShapeDtypeStruct((NTOK_, DIM_), jnp.float32),
        mesh=plsc.VectorSubcoreMesh(core_axis_name="c", subcore_axis_name="s"),
        scratch_types=[
            pltpu.VMEM((_ROWS_PER_W_ // _IDX_CHUNK_, _IDX_CHUNK_), jnp.int32),
            pltpu.VMEM((_ROWS_PER_W_, DIM_), jnp.float32),
            pltpu.SemaphoreType.DMA,
        ],
    )


def _finish_body(x_ref, q_ref, qst_ref, diff_ref):
    i = pl.program_id(0)
    x = x_ref[...]
    d = q_ref[...] - x
    qst_ref[...] = x + d

    @pl.when(i == 0)
    def _init():
        diff_ref[...] = jnp.zeros((1, 1), jnp.float32)

    diff_ref[...] += jnp.sum(d * d).reshape(1, 1)

    @pl.when(i == NBLK_ - 1)
    def _final():
        diff_ref[...] = diff_ref[...] * jnp.float32(1.0 / (NTOK_ * DIM_))


def _finish(x, q):
    return pl.pallas_call(
        _finish_body,
        grid=(NBLK_,),
        in_specs=[
            pl.BlockSpec((TOK_BLK_, DIM_), lambda i: (i, 0)),
            pl.BlockSpec((TOK_BLK_, DIM_), lambda i: (i, 0)),
        ],
        out_specs=[
            pl.BlockSpec((TOK_BLK_, DIM_), lambda i: (i, 0)),
            pl.BlockSpec((1, 1), lambda i: (0, 0)),
        ],
        out_shape=[
            jax.ShapeDtypeStruct((NTOK_, DIM_), jnp.float32),
            jax.ShapeDtypeStruct((1, 1), jnp.float32),
        ],
        compiler_params=pltpu.CompilerParams(
            dimension_semantics=("arbitrary",),
        ),
    )(x, q)


def kernel(input, embed):
    x = input.reshape(NTOK_, DIM_)
    ind3, table = _compute_ind(x, embed)
    ind = ind3.reshape(NTOK_)
    idx2 = ind.reshape(NTOK_ // _IDX_CHUNK_, _IDX_CHUNK_)
    q = _make_sc_gather()(table, idx2)
    qst, diff = _finish(x, q)
    return (qst.reshape(input.shape),
            diff.reshape(()),
            jnp.zeros(1, dtype=jnp.float32),
            ind.reshape(input.shape[:-1]))
